# baseline (speedup 1.0000x reference)
import functools

import jax
import jax.numpy as jnp
from jax.experimental import pallas as pl
from jax.experimental.pallas import tpu as pltpu


def _round_up(x: int, m: int) -> int:
    return ((x + m - 1) // m) * m


def _static_pow(a, n: int):
    result = None
    base = a
    e = n
    while e:
        if e & 1:
            result = base if result is None else result * base
        e >>= 1
        if e:
            base = base * base
    return result


def _mlp_kernel(x_ref, w_ref, b_ref, o_ref, *, n_linear: int, n_pow: int, mc: int):
    tb = x_ref.shape[0]
    chunks = list(range(0, tb, mc))
    h = [x_ref[pl.ds(s, min(mc, tb - s)), :].astype(jnp.float8_e4m3fn) for s in chunks]
    for l in range(n_linear):
        last = l == n_linear - 1
        for ci, s in enumerate(chunks):
            acc = jnp.dot(h[ci], w_ref[l], preferred_element_type=jnp.float32)
            acc = jnp.maximum(acc + b_ref[l], 0.0)
            p = _static_pow(acc, n_pow)
            if last:
                rows = pl.ds(s, min(mc, tb - s))
                o_ref[rows, :] = x_ref[rows, :] + p
            else:
                h[ci] = p.astype(jnp.float8_e4m3fn)


def _drm(x, w_stack, b_stack, *, n_pow: int):
    n_linear, Wp, _ = w_stack.shape
    B, W = x.shape

    tb = min(2048, _round_up(B, 8))
    Bp = _round_up(B, tb)
    n_tiles = Bp // tb
    mc = min(256, tb)

    if (Bp, Wp) != (B, W):
        x = jnp.pad(x, ((0, Bp - B), (0, Wp - W)))
    w_fp8 = w_stack.astype(jnp.float8_e4m3fn)

    out = pl.pallas_call(
        functools.partial(_mlp_kernel, n_linear=n_linear, n_pow=n_pow, mc=mc),
        out_shape=jax.ShapeDtypeStruct((Bp, Wp), jnp.float32),
        grid=(n_tiles,),
        in_specs=[
            pl.BlockSpec((tb, Wp), lambda i: (i, 0)),
            pl.BlockSpec((n_linear, Wp, Wp), lambda i: (0, 0, 0)),
            pl.BlockSpec((n_linear, 1, Wp), lambda i: (0, 0, 0)),
        ],
        out_specs=pl.BlockSpec((tb, Wp), lambda i: (i, 0)),
        compiler_params=pltpu.CompilerParams(
            dimension_semantics=("arbitrary",),
            vmem_limit_bytes=56 << 20,
        ),
    )(x, w_fp8, b_stack)
    return out[:B, :W]


def kernel(x, w_stack, b_stack):
    return _drm(x, w_stack, b_stack, n_pow=3)


# speedup vs baseline: 1.1466x; 1.1466x over previous
import functools

import jax
import jax.numpy as jnp
from jax.experimental import pallas as pl
from jax.experimental.pallas import tpu as pltpu


def _round_up(x: int, m: int) -> int:
    return ((x + m - 1) // m) * m


def _static_pow(a, n: int):
    result = None
    base = a
    e = n
    while e:
        if e & 1:
            result = base if result is None else result * base
        e >>= 1
        if e:
            base = base * base
    return result


def _mlp_kernel(x_ref, w_ref, b_ref, o_ref, wb_ref,
                *, n_linear: int, n_pow: int, mc: int):
    @pl.when(pl.program_id(0) == 0)
    def _():
        for l in range(n_linear):
            wb_ref[l] = w_ref[l].astype(jnp.float8_e4m3fn)

    tb = x_ref.shape[0]
    chunks = list(range(0, tb, mc))
    h = [x_ref[pl.ds(s, min(mc, tb - s)), :].astype(jnp.float8_e4m3fn) for s in chunks]
    for l in range(n_linear):
        last = l == n_linear - 1
        for ci, s in enumerate(chunks):
            acc = jnp.dot(h[ci], wb_ref[l], preferred_element_type=jnp.float32)
            acc = jnp.maximum(acc + b_ref[l], 0.0)
            p = _static_pow(acc, n_pow)
            if last:
                rows = pl.ds(s, min(mc, tb - s))
                o_ref[rows, :] = x_ref[rows, :] + p
            else:
                h[ci] = p.astype(jnp.float8_e4m3fn)


def _drm(x, w_stack, b_stack, *, n_pow: int):
    n_linear, Wp, _ = w_stack.shape
    B, W = x.shape

    tb = min(1024, _round_up(B, 8))
    Bp = _round_up(B, tb)
    n_tiles = Bp // tb
    mc = min(256, tb)

    if (Bp, Wp) != (B, W):
        x = jnp.pad(x, ((0, Bp - B), (0, Wp - W)))

    out = pl.pallas_call(
        functools.partial(_mlp_kernel, n_linear=n_linear, n_pow=n_pow, mc=mc),
        out_shape=jax.ShapeDtypeStruct((Bp, Wp), jnp.float32),
        grid=(n_tiles,),
        in_specs=[
            pl.BlockSpec((tb, Wp), lambda i: (i, 0)),
            pl.BlockSpec((n_linear, Wp, Wp), lambda i: (0, 0, 0)),
            pl.BlockSpec((n_linear, 1, Wp), lambda i: (0, 0, 0)),
        ],
        out_specs=pl.BlockSpec((tb, Wp), lambda i: (i, 0)),
        scratch_shapes=[
            pltpu.VMEM((n_linear, Wp, Wp), jnp.float8_e4m3fn),
        ],
        compiler_params=pltpu.CompilerParams(
            dimension_semantics=("arbitrary",),
            vmem_limit_bytes=56 << 20,
        ),
    )(x, w_stack, b_stack)
    return out[:B, :W]


def kernel(x, w_stack, b_stack):
    return _drm(x, w_stack, b_stack, n_pow=3)
